# Initial kernel scaffold; baseline (speedup 1.0000x reference)
import jax
import jax.numpy as jnp
from jax.experimental import pallas as pl


def kernel(x, edge_index, W1, b1, W2):
    raise NotImplementedError("write your pallas kernel here")


# speedup vs baseline: 12.8985x; 12.8985x over previous
import functools

import jax
import jax.numpy as jnp
from jax import lax
from jax.experimental import pallas as pl
from jax.experimental.pallas import tpu as pltpu
from jax.experimental.pallas import tpu_sc as plsc

N = 10000
E = 320000
D = 128

NC = 2
NS = 16
CH = 80
E_PER_TILE = E // (NC * NS)
N_CHUNKS = E_PER_TILE // CH
ROWS_PER_TILE = N // NS
ZROWS = 125
OROWS = 200

_mesh = plsc.VectorSubcoreMesh(core_axis_name="c", subcore_axis_name="s")


def _zero_f32_2d(buf, nrows):
    def body(r, carry):
        for j in range(D // 16):
            buf[r, pl.ds(j * 16, 16)] = jnp.zeros((16,), jnp.float32)
        return carry
    lax.fori_loop(0, nrows, body, 0)


@functools.partial(
    pl.kernel,
    mesh=_mesh,
    out_type=jax.ShapeDtypeStruct((NC, N, D), jnp.float32),
    scratch_types=[
        pltpu.VMEM_SHARED((N, D), jnp.float32),
        pltpu.VMEM((CH,), jnp.int32),
        pltpu.VMEM((1, CH), jnp.int32),
        pltpu.VMEM((CH, D), jnp.float32),
        pltpu.VMEM((OROWS, D), jnp.float32),
        pltpu.SemaphoreType.DMA,
    ],
)
def _sc_aggregate(src_hbm, dst_hbm, g_hbm, out_hbm, acc, src_v, dst_v, rows_v, zbuf, sem):
    c = lax.axis_index("c")
    s = lax.axis_index("s")

    _zero_f32_2d(zbuf, OROWS)
    for t in range(ROWS_PER_TILE // ZROWS):
        pltpu.sync_copy(zbuf.at[pl.ds(0, ZROWS)],
                        acc.at[pl.ds(s * ROWS_PER_TILE + t * ZROWS, ZROWS)])
    plsc.subcore_barrier()

    def chunk(i, carry):
        base = c * (E // NC) + s * E_PER_TILE + i * CH
        pltpu.sync_copy(src_hbm.at[pl.ds(base, CH)], src_v)
        pltpu.sync_copy(dst_hbm.at[pl.ds(base, CH)], dst_v.at[0])
        pltpu.async_copy(g_hbm.at[src_v], rows_v, sem).wait()
        pltpu.sync_copy(rows_v, acc.at[dst_v.at[0]], add=True)
        return carry
    lax.fori_loop(0, N_CHUNKS, chunk, 0)
    plsc.subcore_barrier()

    @pl.when(s < 10)
    def _():
        for t in range(1000 // OROWS):
            r0 = s * 1000 + t * OROWS
            pltpu.sync_copy(acc.at[pl.ds(r0, OROWS)], zbuf)
            pltpu.sync_copy(zbuf, out_hbm.at[c, pl.ds(r0, OROWS)])


@functools.partial(
    pl.kernel,
    mesh=_mesh,
    out_type=jax.ShapeDtypeStruct((NC * N,), jnp.float32),
    scratch_types=[
        pltpu.VMEM_SHARED((N,), jnp.float32),
        pltpu.VMEM((1, CH), jnp.int32),
        pltpu.VMEM((CH,), jnp.float32),
        pltpu.VMEM((2000,), jnp.float32),
    ],
)
def _sc_degree(dst_hbm, out_hbm, dacc, dst_v, ones_v, zbuf):
    c = lax.axis_index("c")
    s = lax.axis_index("s")

    def zbody(i, carry):
        zbuf[pl.ds(i * 16, 16)] = jnp.zeros((16,), jnp.float32)
        return carry
    lax.fori_loop(0, 125, zbody, 0)
    for j in range(CH // 16):
        ones_v[pl.ds(j * 16, 16)] = jnp.ones((16,), jnp.float32)

    @pl.when(s == 0)
    def _():
        for t in range(N // 2000):
            pltpu.sync_copy(zbuf, dacc.at[pl.ds(t * 2000, 2000)])
    plsc.subcore_barrier()

    def chunk(i, carry):
        base = c * (E // NC) + s * E_PER_TILE + i * CH
        pltpu.sync_copy(dst_hbm.at[pl.ds(base, CH)], dst_v.at[0])
        pltpu.sync_copy(ones_v, dacc.at[dst_v.at[0]], add=True)
        return carry
    lax.fori_loop(0, N_CHUNKS, chunk, 0)
    plsc.subcore_barrier()

    @pl.when(s < 10)
    def _():
        pltpu.sync_copy(dacc.at[pl.ds(s * 1000, 1000)], zbuf.at[pl.ds(0, 1000)])
        pltpu.sync_copy(zbuf.at[pl.ds(0, 1000)],
                        out_hbm.at[pl.ds(c * N + s * 1000, 1000)])


BR = 1000


def _dinv_block(degp_ref):
    return lax.rsqrt(degp_ref[0] + degp_ref[1] + 1.0)


def _t1_body(x_ref, w_ref, degp_ref, g_ref):
    dinv = _dinv_block(degp_ref)
    h = jnp.dot(x_ref[...], w_ref[...],
                preferred_element_type=jnp.float32,
                precision=lax.Precision.HIGHEST)
    g_ref[...] = h * dinv


def _t2_body(accp_ref, g1_ref, degp_ref, b1_ref, w2_ref, g2_ref):
    dinv = _dinv_block(degp_ref)
    ssum = accp_ref[0] + accp_ref[1] + g1_ref[...]
    z = jnp.maximum(ssum * dinv + b1_ref[...], 0.0)
    h2 = jnp.dot(z, w2_ref[...],
                 preferred_element_type=jnp.float32,
                 precision=lax.Precision.HIGHEST)
    g2_ref[...] = h2 * dinv


def _t3_body(accp_ref, g2_ref, degp_ref, out_ref):
    dinv = _dinv_block(degp_ref)
    y = (accp_ref[0] + accp_ref[1] + g2_ref[...]) * dinv
    m = jnp.max(y, axis=1, keepdims=True)
    lse = jnp.log(jnp.sum(jnp.exp(y - m), axis=1, keepdims=True)) + m
    out_ref[...] = y - lse


_deg_spec = pl.BlockSpec((NC, BR, 1), lambda i: (0, i, 0))
_row_spec = pl.BlockSpec((BR, D), lambda i: (i, 0))
_acc_spec = pl.BlockSpec((NC, BR, D), lambda i: (0, i, 0))
_w_spec = pl.BlockSpec((D, D), lambda i: (0, 0))

_t1 = pl.pallas_call(
    _t1_body,
    grid=(N // BR,),
    in_specs=[_row_spec, _w_spec, _deg_spec],
    out_specs=_row_spec,
    out_shape=jax.ShapeDtypeStruct((N, D), jnp.float32),
)

_t2 = pl.pallas_call(
    _t2_body,
    grid=(N // BR,),
    in_specs=[_acc_spec, _row_spec, _deg_spec,
              pl.BlockSpec((1, D), lambda i: (0, 0)), _w_spec],
    out_specs=_row_spec,
    out_shape=jax.ShapeDtypeStruct((N, D), jnp.float32),
)

_t3 = pl.pallas_call(
    _t3_body,
    grid=(N // BR,),
    in_specs=[_acc_spec, _row_spec, _deg_spec],
    out_specs=_row_spec,
    out_shape=jax.ShapeDtypeStruct((N, D), jnp.float32),
)


def kernel(x, edge_index, W1, b1, W2):
    src = edge_index[0]
    dst = edge_index[1]
    degp = _sc_degree(dst).reshape(NC, N, 1)
    g1 = _t1(x, W1, degp)
    acc1 = _sc_aggregate(src, dst, g1)
    g2 = _t2(acc1, g1, degp, b1.reshape(1, D), W2)
    acc2 = _sc_aggregate(src, dst, g2)
    return _t3(acc2, g2, degp)


# speedup vs baseline: 22.0345x; 1.7083x over previous
import functools

import jax
import jax.numpy as jnp
from jax import lax
from jax.experimental import pallas as pl
from jax.experimental.pallas import tpu as pltpu
from jax.experimental.pallas import tpu_sc as plsc

N = 10000
E = 320000
D = 128

NC = 2
NS = 16
CH = 40
E_PER_TILE = E // (NC * NS)
N_CHUNKS = E_PER_TILE // CH
ROWS_PER_TILE = N // NS
ZROWS = 25
OROWS = 40
SD = 25

_mesh = plsc.VectorSubcoreMesh(core_axis_name="c", subcore_axis_name="s")


def _zero_f32_2d(buf, nrows):
    def body(r, carry):
        for j in range(D // 16):
            buf[r, pl.ds(j * 16, 16)] = jnp.zeros((16,), jnp.float32)
        return carry
    lax.fori_loop(0, nrows, body, 0)


S = 5
NSB = N_CHUNKS // S


@functools.partial(
    pl.kernel,
    mesh=_mesh,
    out_type=jax.ShapeDtypeStruct((NC, N, D), jnp.float32),
    scratch_types=[
        pltpu.VMEM_SHARED((N, D), jnp.float32),
        pltpu.VMEM((S * CH,), jnp.int32),
        pltpu.VMEM((S, CH), jnp.int32),
        pltpu.VMEM((S, CH, D), jnp.float32),
        pltpu.VMEM((OROWS, D), jnp.float32),
        pltpu.SemaphoreType.DMA,
        *([pltpu.SemaphoreType.DMA] * S),
        pltpu.SemaphoreType.DMA,
    ],
)
def _sc_aggregate(src_hbm, dst_hbm, g_hbm, out_hbm, acc, src_blk, dst_blk,
                  rows, zbuf, isem, *gssems):
    gsems, ssem = gssems[:S], gssems[S]
    c = lax.axis_index("c")
    s = lax.axis_index("s")

    _zero_f32_2d(zbuf, OROWS)
    for t in range(ROWS_PER_TILE // ZROWS):
        pltpu.sync_copy(zbuf.at[pl.ds(0, ZROWS)],
                        acc.at[pl.ds(s * ROWS_PER_TILE + t * ZROWS, ZROWS)])
    plsc.subcore_barrier()

    tile_base = c * (E // NC) + s * E_PER_TILE

    def sblock(k, carry):
        base = tile_base + k * (S * CH)
        hi = pltpu.async_copy(src_hbm.at[pl.ds(base, S * CH)], src_blk, isem)
        hds = [pltpu.async_copy(dst_hbm.at[pl.ds(base + j * CH, CH)],
                                dst_blk.at[j], isem) for j in range(S)]
        hi.wait()
        for h in hds:
            h.wait()
        ghs = [pltpu.async_copy(g_hbm.at[src_blk.at[pl.ds(j * CH, CH)]],
                                rows.at[j], gsems[j]) for j in range(S)]
        shs = []
        for j in range(S):
            ghs[j].wait()
            shs.append(pltpu.async_copy(rows.at[j], acc.at[dst_blk.at[j]],
                                        ssem, add=True))
        for h in shs:
            h.wait()
        return carry
    lax.fori_loop(0, NSB, sblock, 0)
    plsc.subcore_barrier()

    @pl.when(s < 10)
    def _():
        for t in range(1000 // OROWS):
            r0 = s * 1000 + t * OROWS
            pltpu.sync_copy(acc.at[pl.ds(r0, OROWS)], zbuf)
            pltpu.sync_copy(zbuf, out_hbm.at[c, pl.ds(r0, OROWS)])


@functools.partial(
    pl.kernel,
    mesh=_mesh,
    out_type=jax.ShapeDtypeStruct((NC * N,), jnp.float32),
    scratch_types=[
        pltpu.VMEM_SHARED((N,), jnp.float32),
        pltpu.VMEM((SD, CH), jnp.int32),
        pltpu.VMEM((48,), jnp.float32),
        pltpu.VMEM((2000,), jnp.float32),
        pltpu.SemaphoreType.DMA,
        pltpu.SemaphoreType.DMA,
    ],
)
def _sc_degree(dst_hbm, out_hbm, dacc, dst_blk, ones_v, zbuf, isem, ssem):
    c = lax.axis_index("c")
    s = lax.axis_index("s")

    def zbody(i, carry):
        zbuf[pl.ds(i * 16, 16)] = jnp.zeros((16,), jnp.float32)
        return carry
    lax.fori_loop(0, 125, zbody, 0)
    for j in range(3):
        ones_v[pl.ds(j * 16, 16)] = jnp.ones((16,), jnp.float32)

    @pl.when(s == 0)
    def _():
        for t in range(N // 2000):
            pltpu.sync_copy(zbuf, dacc.at[pl.ds(t * 2000, 2000)])
    plsc.subcore_barrier()

    tile_base = c * (E // NC) + s * E_PER_TILE

    def sblock(k, carry):
        base = tile_base + k * (SD * CH)
        hds = [pltpu.async_copy(dst_hbm.at[pl.ds(base + j * CH, CH)],
                                dst_blk.at[j], isem) for j in range(SD)]
        for h in hds:
            h.wait()
        shs = [pltpu.async_copy(ones_v.at[pl.ds(0, CH)], dacc.at[dst_blk.at[j]],
                                ssem, add=True)
               for j in range(SD)]
        for h in shs:
            h.wait()
        return carry
    lax.fori_loop(0, N_CHUNKS // SD, sblock, 0)
    plsc.subcore_barrier()

    @pl.when(s < 10)
    def _():
        pltpu.sync_copy(dacc.at[pl.ds(s * 1000, 1000)], zbuf.at[pl.ds(0, 1000)])
        pltpu.sync_copy(zbuf.at[pl.ds(0, 1000)],
                        out_hbm.at[pl.ds(c * N + s * 1000, 1000)])


BR = 1000


def _dinv_block(degp_ref):
    return lax.rsqrt(degp_ref[0] + degp_ref[1] + 1.0)


def _t1_body(x_ref, w_ref, degp_ref, g_ref):
    dinv = _dinv_block(degp_ref)
    h = jnp.dot(x_ref[...], w_ref[...],
                preferred_element_type=jnp.float32,
                precision=lax.Precision.HIGHEST)
    g_ref[...] = h * dinv


def _t2_body(accp_ref, g1_ref, degp_ref, b1_ref, w2_ref, g2_ref):
    dinv = _dinv_block(degp_ref)
    ssum = accp_ref[0] + accp_ref[1] + g1_ref[...]
    z = jnp.maximum(ssum * dinv + b1_ref[...], 0.0)
    h2 = jnp.dot(z, w2_ref[...],
                 preferred_element_type=jnp.float32,
                 precision=lax.Precision.HIGHEST)
    g2_ref[...] = h2 * dinv


def _t3_body(accp_ref, g2_ref, degp_ref, out_ref):
    dinv = _dinv_block(degp_ref)
    y = (accp_ref[0] + accp_ref[1] + g2_ref[...]) * dinv
    m = jnp.max(y, axis=1, keepdims=True)
    lse = jnp.log(jnp.sum(jnp.exp(y - m), axis=1, keepdims=True)) + m
    out_ref[...] = y - lse


_deg_spec = pl.BlockSpec((NC, BR, 1), lambda i: (0, i, 0))
_row_spec = pl.BlockSpec((BR, D), lambda i: (i, 0))
_acc_spec = pl.BlockSpec((NC, BR, D), lambda i: (0, i, 0))
_w_spec = pl.BlockSpec((D, D), lambda i: (0, 0))

_t1 = pl.pallas_call(
    _t1_body,
    grid=(N // BR,),
    in_specs=[_row_spec, _w_spec, _deg_spec],
    out_specs=_row_spec,
    out_shape=jax.ShapeDtypeStruct((N, D), jnp.float32),
)

_t2 = pl.pallas_call(
    _t2_body,
    grid=(N // BR,),
    in_specs=[_acc_spec, _row_spec, _deg_spec,
              pl.BlockSpec((1, D), lambda i: (0, 0)), _w_spec],
    out_specs=_row_spec,
    out_shape=jax.ShapeDtypeStruct((N, D), jnp.float32),
)

_t3 = pl.pallas_call(
    _t3_body,
    grid=(N // BR,),
    in_specs=[_acc_spec, _row_spec, _deg_spec],
    out_specs=_row_spec,
    out_shape=jax.ShapeDtypeStruct((N, D), jnp.float32),
)


def kernel(x, edge_index, W1, b1, W2):
    src = edge_index[0]
    dst = edge_index[1]
    degp = _sc_degree(dst).reshape(NC, N, 1)
    g1 = _t1(x, W1, degp)
    acc1 = _sc_aggregate(src, dst, g1)
    g2 = _t2(acc1, g1, degp, b1.reshape(1, D), W2)
    acc2 = _sc_aggregate(src, dst, g2)
    return _t3(acc2, g2, degp)


# speedup vs baseline: 29.7770x; 1.3514x over previous
import functools

import jax
import jax.numpy as jnp
from jax import lax
from jax.experimental import pallas as pl
from jax.experimental.pallas import tpu as pltpu
from jax.experimental.pallas import tpu_sc as plsc

N = 10000
E = 320000
D = 128

NC = 2
NS = 16
CH = 40
E_PER_TILE = E // (NC * NS)
N_CHUNKS = E_PER_TILE // CH
ROWS_PER_TILE = N // NS
ZROWS = 25
OROWS = 40
SD = 25

_mesh = plsc.VectorSubcoreMesh(core_axis_name="c", subcore_axis_name="s")


def _zero_f32_2d(buf, nrows):
    def body(r, carry):
        for j in range(D // 16):
            buf[r, pl.ds(j * 16, 16)] = jnp.zeros((16,), jnp.float32)
        return carry
    lax.fori_loop(0, nrows, body, 0)


S = 5
NSB = N_CHUNKS // S


@functools.partial(
    pl.kernel,
    mesh=_mesh,
    out_type=jax.ShapeDtypeStruct((NC, N, D), jnp.float32),
    scratch_types=[
        pltpu.VMEM_SHARED((N, D), jnp.float32),
        pltpu.VMEM((2, S, CH), jnp.int32),
        pltpu.VMEM((2, S, CH), jnp.int32),
        pltpu.VMEM((S, CH, D), jnp.float32),
        pltpu.VMEM((OROWS, D), jnp.float32),
        pltpu.SemaphoreType.DMA,
        *([pltpu.SemaphoreType.DMA] * S),
        *([pltpu.SemaphoreType.DMA] * S),
    ],
)
def _sc_aggregate(src_hbm, dst_hbm, g_hbm, out_hbm, acc, src_blk, dst_blk,
                  rows, zbuf, isem, *gssems):
    gsems, ssems = gssems[:S], gssems[S:]
    c = lax.axis_index("c")
    s = lax.axis_index("s")

    _zero_f32_2d(zbuf, OROWS)
    for t in range(ROWS_PER_TILE // ZROWS):
        pltpu.sync_copy(zbuf.at[pl.ds(0, ZROWS)],
                        acc.at[pl.ds(s * ROWS_PER_TILE + t * ZROWS, ZROWS)])
    plsc.subcore_barrier()

    tile_base = c * (E // NC) + s * E_PER_TILE

    def issue_idx(k, b):
        base = tile_base + k * (S * CH)
        for j in range(S):
            pltpu.async_copy(src_hbm.at[pl.ds(base + j * CH, CH)],
                             src_blk.at[b, j], isem)
            pltpu.async_copy(dst_hbm.at[pl.ds(base + j * CH, CH)],
                             dst_blk.at[b, j], isem)

    issue_idx(0, 0)

    def sblock(k, carry):
        b = lax.rem(k, 2)
        base = tile_base + k * (S * CH)
        for j in range(S):
            pltpu.make_async_copy(src_hbm.at[pl.ds(base + j * CH, CH)],
                                  src_blk.at[b, j], isem).wait()
            pltpu.make_async_copy(dst_hbm.at[pl.ds(base + j * CH, CH)],
                                  dst_blk.at[b, j], isem).wait()

        @pl.when(k + 1 < NSB)
        def _():
            issue_idx(k + 1, 1 - b)

        ghs = []
        for j in range(S):
            @pl.when(k > 0)
            def _(j=j):
                pltpu.make_async_copy(rows.at[j], acc.at[dst_blk.at[b, j]],
                                      ssems[j]).wait()
            ghs.append(pltpu.async_copy(
                g_hbm.at[src_blk.at[b, j]], rows.at[j], gsems[j]))
        for j in range(S):
            ghs[j].wait()
            pltpu.async_copy(rows.at[j], acc.at[dst_blk.at[b, j]],
                             ssems[j], add=True)
        return carry
    lax.fori_loop(0, NSB, sblock, 0)

    for j in range(S):
        pltpu.make_async_copy(rows.at[j], acc.at[dst_blk.at[0, j]],
                              ssems[j]).wait()
    plsc.subcore_barrier()

    @pl.when(s < 10)
    def _():
        for t in range(1000 // OROWS):
            r0 = s * 1000 + t * OROWS
            pltpu.sync_copy(acc.at[pl.ds(r0, OROWS)], zbuf)
            pltpu.sync_copy(zbuf, out_hbm.at[c, pl.ds(r0, OROWS)])


@functools.partial(
    pl.kernel,
    mesh=_mesh,
    out_type=jax.ShapeDtypeStruct((NC * N,), jnp.float32),
    scratch_types=[
        pltpu.VMEM_SHARED((N,), jnp.float32),
        pltpu.VMEM((SD, CH), jnp.int32),
        pltpu.VMEM((48,), jnp.float32),
        pltpu.VMEM((2000,), jnp.float32),
        pltpu.SemaphoreType.DMA,
        pltpu.SemaphoreType.DMA,
    ],
)
def _sc_degree(dst_hbm, out_hbm, dacc, dst_blk, ones_v, zbuf, isem, ssem):
    c = lax.axis_index("c")
    s = lax.axis_index("s")

    def zbody(i, carry):
        zbuf[pl.ds(i * 16, 16)] = jnp.zeros((16,), jnp.float32)
        return carry
    lax.fori_loop(0, 125, zbody, 0)
    for j in range(3):
        ones_v[pl.ds(j * 16, 16)] = jnp.ones((16,), jnp.float32)

    @pl.when(s == 0)
    def _():
        for t in range(N // 2000):
            pltpu.sync_copy(zbuf, dacc.at[pl.ds(t * 2000, 2000)])
    plsc.subcore_barrier()

    tile_base = c * (E // NC) + s * E_PER_TILE

    def sblock(k, carry):
        base = tile_base + k * (SD * CH)
        hds = [pltpu.async_copy(dst_hbm.at[pl.ds(base + j * CH, CH)],
                                dst_blk.at[j], isem) for j in range(SD)]
        for h in hds:
            h.wait()
        shs = [pltpu.async_copy(ones_v.at[pl.ds(0, CH)], dacc.at[dst_blk.at[j]],
                                ssem, add=True)
               for j in range(SD)]
        for h in shs:
            h.wait()
        return carry
    lax.fori_loop(0, N_CHUNKS // SD, sblock, 0)
    plsc.subcore_barrier()

    @pl.when(s < 10)
    def _():
        pltpu.sync_copy(dacc.at[pl.ds(s * 1000, 1000)], zbuf.at[pl.ds(0, 1000)])
        pltpu.sync_copy(zbuf.at[pl.ds(0, 1000)],
                        out_hbm.at[pl.ds(c * N + s * 1000, 1000)])


BR = 1000


def _dinv_block(degp_ref):
    return lax.rsqrt(degp_ref[0] + degp_ref[1] + 1.0)


def _t1_body(x_ref, w_ref, degp_ref, g_ref):
    dinv = _dinv_block(degp_ref)
    h = jnp.dot(x_ref[...], w_ref[...],
                preferred_element_type=jnp.float32,
                precision=lax.Precision.DEFAULT)
    g_ref[...] = h * dinv


def _t2_body(accp_ref, g1_ref, degp_ref, b1_ref, w2_ref, g2_ref):
    dinv = _dinv_block(degp_ref)
    ssum = accp_ref[0] + accp_ref[1] + g1_ref[...]
    z = jnp.maximum(ssum * dinv + b1_ref[...], 0.0)
    h2 = jnp.dot(z, w2_ref[...],
                 preferred_element_type=jnp.float32,
                 precision=lax.Precision.DEFAULT)
    g2_ref[...] = h2 * dinv


def _t3_body(accp_ref, g2_ref, degp_ref, out_ref):
    dinv = _dinv_block(degp_ref)
    y = (accp_ref[0] + accp_ref[1] + g2_ref[...]) * dinv
    m = jnp.max(y, axis=1, keepdims=True)
    lse = jnp.log(jnp.sum(jnp.exp(y - m), axis=1, keepdims=True)) + m
    out_ref[...] = y - lse


_deg_spec = pl.BlockSpec((NC, BR, 1), lambda i: (0, i, 0))
_row_spec = pl.BlockSpec((BR, D), lambda i: (i, 0))
_acc_spec = pl.BlockSpec((NC, BR, D), lambda i: (0, i, 0))
_w_spec = pl.BlockSpec((D, D), lambda i: (0, 0))

_t1 = pl.pallas_call(
    _t1_body,
    grid=(N // BR,),
    in_specs=[_row_spec, _w_spec, _deg_spec],
    out_specs=_row_spec,
    out_shape=jax.ShapeDtypeStruct((N, D), jnp.float32),
)

_t2 = pl.pallas_call(
    _t2_body,
    grid=(N // BR,),
    in_specs=[_acc_spec, _row_spec, _deg_spec,
              pl.BlockSpec((1, D), lambda i: (0, 0)), _w_spec],
    out_specs=_row_spec,
    out_shape=jax.ShapeDtypeStruct((N, D), jnp.float32),
)

_t3 = pl.pallas_call(
    _t3_body,
    grid=(N // BR,),
    in_specs=[_acc_spec, _row_spec, _deg_spec],
    out_specs=_row_spec,
    out_shape=jax.ShapeDtypeStruct((N, D), jnp.float32),
)


def kernel(x, edge_index, W1, b1, W2):
    src = edge_index[0]
    dst = edge_index[1]
    degp = _sc_degree(dst).reshape(NC, N, 1)
    g1 = _t1(x, W1, degp)
    acc1 = _sc_aggregate(src, dst, g1)
    g2 = _t2(acc1, g1, degp, b1.reshape(1, D), W2)
    acc2 = _sc_aggregate(src, dst, g2)
    return _t3(acc2, g2, degp)


# speedup vs baseline: 31.2091x; 1.0481x over previous
import functools

import jax
import jax.numpy as jnp
from jax import lax
from jax.experimental import pallas as pl
from jax.experimental.pallas import tpu as pltpu
from jax.experimental.pallas import tpu_sc as plsc

N = 10000
E = 320000
D = 128

NC = 2
NS = 16
CH = 80
E_PER_TILE = E // (NC * NS)
N_CHUNKS = E_PER_TILE // CH
ROWS_PER_TILE = N // NS
ZROWS = 25
OROWS = 40
SD = 25

_mesh = plsc.VectorSubcoreMesh(core_axis_name="c", subcore_axis_name="s")


def _zero_f32_2d(buf, nrows):
    def body(r, carry):
        for j in range(D // 16):
            buf[r, pl.ds(j * 16, 16)] = jnp.zeros((16,), jnp.float32)
        return carry
    lax.fori_loop(0, nrows, body, 0)


S = 4
NSB = (N_CHUNKS + S - 1) // S


@functools.partial(
    pl.kernel,
    mesh=_mesh,
    out_type=jax.ShapeDtypeStruct((NC, N, D), jnp.float32),
    scratch_types=[
        pltpu.VMEM_SHARED((N, D), jnp.float32),
        pltpu.VMEM((2, S, CH), jnp.int32),
        pltpu.VMEM((2, S, CH), jnp.int32),
        pltpu.VMEM((S, CH, D), jnp.float32),
        pltpu.VMEM((OROWS, D), jnp.float32),
        pltpu.SemaphoreType.DMA,
        *([pltpu.SemaphoreType.DMA] * S),
        *([pltpu.SemaphoreType.DMA] * S),
    ],
)
def _sc_aggregate(src_hbm, dst_hbm, g_hbm, out_hbm, acc, src_blk, dst_blk,
                  rows, zbuf, isem, *gssems):
    gsems, ssems = gssems[:S], gssems[S:]
    c = lax.axis_index("c")
    s = lax.axis_index("s")

    _zero_f32_2d(zbuf, OROWS)
    for t in range(ROWS_PER_TILE // ZROWS):
        pltpu.sync_copy(zbuf.at[pl.ds(0, ZROWS)],
                        acc.at[pl.ds(s * ROWS_PER_TILE + t * ZROWS, ZROWS)])
    plsc.subcore_barrier()

    tile_base = c * (E // NC) + s * E_PER_TILE

    def issue_idx(k, b):
        base = tile_base + k * (S * CH)
        for j in range(S):
            @pl.when(k * S + j < N_CHUNKS)
            def _(j=j):
                pltpu.async_copy(src_hbm.at[pl.ds(base + j * CH, CH)],
                                 src_blk.at[b, j], isem)
                pltpu.async_copy(dst_hbm.at[pl.ds(base + j * CH, CH)],
                                 dst_blk.at[b, j], isem)

    issue_idx(0, 0)

    def sblock(k, carry):
        b = lax.rem(k, 2)
        base = tile_base + k * (S * CH)
        for j in range(S):
            @pl.when(k * S + j < N_CHUNKS)
            def _(j=j):
                pltpu.make_async_copy(src_hbm.at[pl.ds(base + j * CH, CH)],
                                      src_blk.at[b, j], isem).wait()
                pltpu.make_async_copy(dst_hbm.at[pl.ds(base + j * CH, CH)],
                                      dst_blk.at[b, j], isem).wait()

        for j in range(S):
            @pl.when(k > 0)
            def _(j=j):
                pltpu.make_async_copy(rows.at[j], acc.at[dst_blk.at[b, j]],
                                      ssems[j]).wait()

            @pl.when(k * S + j < N_CHUNKS)
            def _(j=j):
                pltpu.async_copy(
                    g_hbm.at[src_blk.at[b, j]], rows.at[j], gsems[j])

        @pl.when(k + 1 < NSB)
        def _():
            issue_idx(k + 1, 1 - b)
        for j in range(S):
            @pl.when(k * S + j < N_CHUNKS)
            def _(j=j):
                pltpu.make_async_copy(g_hbm.at[src_blk.at[b, j]], rows.at[j],
                                      gsems[j]).wait()
                pltpu.async_copy(rows.at[j], acc.at[dst_blk.at[b, j]],
                                 ssems[j], add=True)
        return carry
    lax.fori_loop(0, NSB, sblock, 0)

    for j in range(S):
        @pl.when((NSB - 1) * S + j < N_CHUNKS)
        def _(j=j):
            pltpu.make_async_copy(rows.at[j], acc.at[dst_blk.at[0, j]],
                                  ssems[j]).wait()
    plsc.subcore_barrier()

    @pl.when(s < 10)
    def _():
        for t in range(1000 // OROWS):
            r0 = s * 1000 + t * OROWS
            pltpu.sync_copy(acc.at[pl.ds(r0, OROWS)], zbuf)
            pltpu.sync_copy(zbuf, out_hbm.at[c, pl.ds(r0, OROWS)])


@functools.partial(
    pl.kernel,
    mesh=_mesh,
    out_type=jax.ShapeDtypeStruct((NC * N,), jnp.float32),
    scratch_types=[
        pltpu.VMEM_SHARED((N,), jnp.float32),
        pltpu.VMEM((SD, CH), jnp.int32),
        pltpu.VMEM((80,), jnp.float32),
        pltpu.VMEM((2000,), jnp.float32),
        pltpu.SemaphoreType.DMA,
        pltpu.SemaphoreType.DMA,
    ],
)
def _sc_degree(dst_hbm, out_hbm, dacc, dst_blk, ones_v, zbuf, isem, ssem):
    c = lax.axis_index("c")
    s = lax.axis_index("s")

    def zbody(i, carry):
        zbuf[pl.ds(i * 16, 16)] = jnp.zeros((16,), jnp.float32)
        return carry
    lax.fori_loop(0, 125, zbody, 0)
    for j in range(5):
        ones_v[pl.ds(j * 16, 16)] = jnp.ones((16,), jnp.float32)

    @pl.when(s == 0)
    def _():
        for t in range(N // 2000):
            pltpu.sync_copy(zbuf, dacc.at[pl.ds(t * 2000, 2000)])
    plsc.subcore_barrier()

    tile_base = c * (E // NC) + s * E_PER_TILE

    def sblock(k, carry):
        base = tile_base + k * (SD * CH)
        hds = [pltpu.async_copy(dst_hbm.at[pl.ds(base + j * CH, CH)],
                                dst_blk.at[j], isem) for j in range(SD)]
        for h in hds:
            h.wait()
        shs = [pltpu.async_copy(ones_v.at[pl.ds(0, CH)], dacc.at[dst_blk.at[j]],
                                ssem, add=True)
               for j in range(SD)]
        for h in shs:
            h.wait()
        return carry
    lax.fori_loop(0, N_CHUNKS // SD, sblock, 0)
    plsc.subcore_barrier()

    @pl.when(s < 10)
    def _():
        pltpu.sync_copy(dacc.at[pl.ds(s * 1000, 1000)], zbuf.at[pl.ds(0, 1000)])
        pltpu.sync_copy(zbuf.at[pl.ds(0, 1000)],
                        out_hbm.at[pl.ds(c * N + s * 1000, 1000)])


BR = 2000


def _dinv_block(degp_ref):
    return lax.rsqrt(degp_ref[0] + degp_ref[1] + 1.0)


def _t0_body(x_ref, w_ref, h_ref):
    h_ref[...] = jnp.dot(x_ref[...], w_ref[...],
                         preferred_element_type=jnp.float32,
                         precision=lax.Precision.DEFAULT)


def _t1_body(h_ref, degp_ref, g_ref):
    g_ref[...] = h_ref[...] * _dinv_block(degp_ref)


def _t2_body(accp_ref, g1_ref, degp_ref, b1_ref, w2_ref, g2_ref):
    dinv = _dinv_block(degp_ref)
    ssum = accp_ref[0] + accp_ref[1] + g1_ref[...]
    z = jnp.maximum(ssum * dinv + b1_ref[...], 0.0)
    h2 = jnp.dot(z, w2_ref[...],
                 preferred_element_type=jnp.float32,
                 precision=lax.Precision.DEFAULT)
    g2_ref[...] = h2 * dinv


def _t3_body(accp_ref, g2_ref, degp_ref, out_ref):
    dinv = _dinv_block(degp_ref)
    y = (accp_ref[0] + accp_ref[1] + g2_ref[...]) * dinv
    m = jnp.max(y, axis=1, keepdims=True)
    lse = jnp.log(jnp.sum(jnp.exp(y - m), axis=1, keepdims=True)) + m
    out_ref[...] = y - lse


_deg_spec = pl.BlockSpec((NC, BR, 1), lambda i: (0, i, 0))
_row_spec = pl.BlockSpec((BR, D), lambda i: (i, 0))
_acc_spec = pl.BlockSpec((NC, BR, D), lambda i: (0, i, 0))
_w_spec = pl.BlockSpec((D, D), lambda i: (0, 0))

_t0 = pl.pallas_call(
    _t0_body,
    grid=(N // BR,),
    in_specs=[_row_spec, _w_spec],
    out_specs=_row_spec,
    out_shape=jax.ShapeDtypeStruct((N, D), jnp.float32),
)

_t1 = pl.pallas_call(
    _t1_body,
    grid=(N // BR,),
    in_specs=[_row_spec, _deg_spec],
    out_specs=_row_spec,
    out_shape=jax.ShapeDtypeStruct((N, D), jnp.float32),
)

_t2 = pl.pallas_call(
    _t2_body,
    grid=(N // BR,),
    in_specs=[_acc_spec, _row_spec, _deg_spec,
              pl.BlockSpec((1, D), lambda i: (0, 0)), _w_spec],
    out_specs=_row_spec,
    out_shape=jax.ShapeDtypeStruct((N, D), jnp.float32),
)

_t3 = pl.pallas_call(
    _t3_body,
    grid=(N // BR,),
    in_specs=[_acc_spec, _row_spec, _deg_spec],
    out_specs=_row_spec,
    out_shape=jax.ShapeDtypeStruct((N, D), jnp.float32),
)


def kernel(x, edge_index, W1, b1, W2):
    src = edge_index[0]
    dst = edge_index[1]
    h1 = _t0(x, W1)
    degp = _sc_degree(dst).reshape(NC, N, 1)
    g1 = _t1(h1, degp)
    acc1 = _sc_aggregate(src, dst, g1)
    g2 = _t2(acc1, g1, degp, b1.reshape(1, D), W2)
    acc2 = _sc_aggregate(src, dst, g2)
    return _t3(acc2, g2, degp)


# speedup vs baseline: 33.0368x; 1.0586x over previous
import functools

import jax
import jax.numpy as jnp
from jax import lax
from jax.experimental import pallas as pl
from jax.experimental.pallas import tpu as pltpu
from jax.experimental.pallas import tpu_sc as plsc

N = 10000
E = 320000
D = 128

NC = 2
NS = 16
CH = 40
E_PER_TILE = E // (NC * NS)
N_CHUNKS = E_PER_TILE // CH
ROWS_PER_TILE = N // NS
ZROWS = 25
OROWS = 40
SD = 25

_mesh = plsc.VectorSubcoreMesh(core_axis_name="c", subcore_axis_name="s")


def _zero_f32_2d(buf, nrows):
    def body(r, carry):
        for j in range(D // 16):
            buf[r, pl.ds(j * 16, 16)] = jnp.zeros((16,), jnp.float32)
        return carry
    lax.fori_loop(0, nrows, body, 0)


R = 8
L = 4
NGRP = (N_CHUNKS + R - 1) // R


@functools.partial(
    pl.kernel,
    mesh=_mesh,
    out_type=jax.ShapeDtypeStruct((NC, N, D), jnp.float32),
    scratch_types=[
        pltpu.VMEM_SHARED((N, D), jnp.float32),
        pltpu.VMEM((3, R, CH), jnp.int32),
        pltpu.VMEM((3, R, CH), jnp.int32),
        pltpu.VMEM((R, CH, D), jnp.float32),
        pltpu.SemaphoreType.DMA,
        *([pltpu.SemaphoreType.DMA] * R),
        *([pltpu.SemaphoreType.DMA] * R),
    ],
)
def _sc_aggregate(src_hbm, dst_hbm, g_hbm, out_hbm, acc, src_blk, dst_blk,
                  rows, isem, *gssems):
    gsems, ssems = gssems[:R], gssems[R:]
    c = lax.axis_index("c")
    s = lax.axis_index("s")

    def zb(r, carry):
        for j in range(D // 16):
            rows[0, r, pl.ds(j * 16, 16)] = jnp.zeros((16,), jnp.float32)
        return carry
    lax.fori_loop(0, ZROWS, zb, 0)
    for t in range(ROWS_PER_TILE // ZROWS):
        pltpu.sync_copy(rows.at[0, pl.ds(0, ZROWS)],
                        acc.at[pl.ds(s * ROWS_PER_TILE + t * ZROWS, ZROWS)])
    plsc.subcore_barrier()

    tile_base = c * (E // NC) + s * E_PER_TILE

    def issue_idx_group(grp, bank):
        for jj in range(R):
            ch = grp * R + jj

            @pl.when(ch < N_CHUNKS)
            def _(jj=jj, ch=ch):
                base = tile_base + ch * CH
                pltpu.async_copy(src_hbm.at[pl.ds(base, CH)],
                                 src_blk.at[bank, jj], isem)
                pltpu.async_copy(dst_hbm.at[pl.ds(base, CH)],
                                 dst_blk.at[bank, jj], isem)

    def drain_idx_group(grp, bank):
        for jj in range(R):
            ch = grp * R + jj

            @pl.when(ch < N_CHUNKS)
            def _(jj=jj, ch=ch):
                base = tile_base + ch * CH
                pltpu.make_async_copy(src_hbm.at[pl.ds(base, CH)],
                                      src_blk.at[bank, jj], isem).wait()
                pltpu.make_async_copy(dst_hbm.at[pl.ds(base, CH)],
                                      dst_blk.at[bank, jj], isem).wait()

    issue_idx_group(0, 0)
    issue_idx_group(1, 1)
    drain_idx_group(0, 0)
    for j in range(L):
        pltpu.async_copy(g_hbm.at[src_blk.at[0, j]], rows.at[j], gsems[j])

    def group_body(G, carry):
        bG = lax.rem(G, 3)
        for j in range(R):
            if j == L:
                issue_idx_group(G + 2, lax.rem(G + 2, 3))
                drain_idx_group(G + 1, lax.rem(G + 1, 3))

            a = G * R + j + L
            ra = (j + L) % R
            bank_a = bG if j < L else lax.rem(G + 1, 3)

            @pl.when(a < N_CHUNKS)
            def _(j=j, a=a, ra=ra, bank_a=bank_a):
                @pl.when(a >= R)
                def _():
                    pltpu.make_async_copy(rows.at[ra], acc.at[dst_blk.at[0, 0]],
                                          ssems[ra]).wait()
                pltpu.async_copy(g_hbm.at[src_blk.at[bank_a, ra]],
                                 rows.at[ra], gsems[ra])

            c0 = G * R + j

            @pl.when(c0 < N_CHUNKS)
            def _(j=j, c0=c0):
                pltpu.make_async_copy(g_hbm.at[src_blk.at[bG, j]], rows.at[j],
                                      gsems[j]).wait()
                pltpu.async_copy(rows.at[j], acc.at[dst_blk.at[bG, j]],
                                 ssems[j], add=True)
        return carry
    lax.fori_loop(0, NGRP, group_body, 0)

    for r in range(R):
        pltpu.make_async_copy(rows.at[r], acc.at[dst_blk.at[0, 0]],
                              ssems[r]).wait()
    plsc.subcore_barrier()

    @pl.when(s < 10)
    def _():
        for t in range(1000 // OROWS):
            r0 = s * 1000 + t * OROWS
            pltpu.sync_copy(acc.at[pl.ds(r0, OROWS)], rows.at[0])
            pltpu.sync_copy(rows.at[0], out_hbm.at[c, pl.ds(r0, OROWS)])


@functools.partial(
    pl.kernel,
    mesh=_mesh,
    out_type=jax.ShapeDtypeStruct((NC * N,), jnp.float32),
    scratch_types=[
        pltpu.VMEM_SHARED((N,), jnp.float32),
        pltpu.VMEM((SD, CH), jnp.int32),
        pltpu.VMEM((80,), jnp.float32),
        pltpu.VMEM((2000,), jnp.float32),
        pltpu.SemaphoreType.DMA,
        pltpu.SemaphoreType.DMA,
    ],
)
def _sc_degree(dst_hbm, out_hbm, dacc, dst_blk, ones_v, zbuf, isem, ssem):
    c = lax.axis_index("c")
    s = lax.axis_index("s")

    def zbody(i, carry):
        zbuf[pl.ds(i * 16, 16)] = jnp.zeros((16,), jnp.float32)
        return carry
    lax.fori_loop(0, 125, zbody, 0)
    for j in range(5):
        ones_v[pl.ds(j * 16, 16)] = jnp.ones((16,), jnp.float32)

    @pl.when(s == 0)
    def _():
        for t in range(N // 2000):
            pltpu.sync_copy(zbuf, dacc.at[pl.ds(t * 2000, 2000)])
    plsc.subcore_barrier()

    tile_base = c * (E // NC) + s * E_PER_TILE

    def sblock(k, carry):
        base = tile_base + k * (SD * CH)
        hds = [pltpu.async_copy(dst_hbm.at[pl.ds(base + j * CH, CH)],
                                dst_blk.at[j], isem) for j in range(SD)]
        for h in hds:
            h.wait()
        shs = [pltpu.async_copy(ones_v.at[pl.ds(0, CH)], dacc.at[dst_blk.at[j]],
                                ssem, add=True)
               for j in range(SD)]
        for h in shs:
            h.wait()
        return carry
    lax.fori_loop(0, N_CHUNKS // SD, sblock, 0)
    plsc.subcore_barrier()

    @pl.when(s < 10)
    def _():
        pltpu.sync_copy(dacc.at[pl.ds(s * 1000, 1000)], zbuf.at[pl.ds(0, 1000)])
        pltpu.sync_copy(zbuf.at[pl.ds(0, 1000)],
                        out_hbm.at[pl.ds(c * N + s * 1000, 1000)])


BR = 2000


def _dinv_block(degp_ref):
    return lax.rsqrt(degp_ref[0] + degp_ref[1] + 1.0)


def _t0_body(x_ref, w_ref, h_ref):
    h_ref[...] = jnp.dot(x_ref[...], w_ref[...],
                         preferred_element_type=jnp.float32,
                         precision=lax.Precision.DEFAULT)


def _t1_body(h_ref, degp_ref, g_ref):
    g_ref[...] = h_ref[...] * _dinv_block(degp_ref)


def _t2_body(accp_ref, g1_ref, degp_ref, b1_ref, w2_ref, g2_ref):
    dinv = _dinv_block(degp_ref)
    ssum = accp_ref[0] + accp_ref[1] + g1_ref[...]
    z = jnp.maximum(ssum * dinv + b1_ref[...], 0.0)
    h2 = jnp.dot(z, w2_ref[...],
                 preferred_element_type=jnp.float32,
                 precision=lax.Precision.DEFAULT)
    g2_ref[...] = h2 * dinv


def _t3_body(accp_ref, g2_ref, degp_ref, out_ref):
    dinv = _dinv_block(degp_ref)
    y = (accp_ref[0] + accp_ref[1] + g2_ref[...]) * dinv
    m = jnp.max(y, axis=1, keepdims=True)
    lse = jnp.log(jnp.sum(jnp.exp(y - m), axis=1, keepdims=True)) + m
    out_ref[...] = y - lse


_deg_spec = pl.BlockSpec((NC, BR, 1), lambda i: (0, i, 0))
_row_spec = pl.BlockSpec((BR, D), lambda i: (i, 0))
_acc_spec = pl.BlockSpec((NC, BR, D), lambda i: (0, i, 0))
_w_spec = pl.BlockSpec((D, D), lambda i: (0, 0))

_t0 = pl.pallas_call(
    _t0_body,
    grid=(N // BR,),
    in_specs=[_row_spec, _w_spec],
    out_specs=_row_spec,
    out_shape=jax.ShapeDtypeStruct((N, D), jnp.float32),
)

_t1 = pl.pallas_call(
    _t1_body,
    grid=(N // BR,),
    in_specs=[_row_spec, _deg_spec],
    out_specs=_row_spec,
    out_shape=jax.ShapeDtypeStruct((N, D), jnp.float32),
)

_t2 = pl.pallas_call(
    _t2_body,
    grid=(N // BR,),
    in_specs=[_acc_spec, _row_spec, _deg_spec,
              pl.BlockSpec((1, D), lambda i: (0, 0)), _w_spec],
    out_specs=_row_spec,
    out_shape=jax.ShapeDtypeStruct((N, D), jnp.float32),
)

_t3 = pl.pallas_call(
    _t3_body,
    grid=(N // BR,),
    in_specs=[_acc_spec, _row_spec, _deg_spec],
    out_specs=_row_spec,
    out_shape=jax.ShapeDtypeStruct((N, D), jnp.float32),
)


def kernel(x, edge_index, W1, b1, W2):
    src = edge_index[0]
    dst = edge_index[1]
    h1 = _t0(x, W1)
    degp = _sc_degree(dst).reshape(NC, N, 1)
    g1 = _t1(h1, degp)
    acc1 = _sc_aggregate(src, dst, g1)
    g2 = _t2(acc1, g1, degp, b1.reshape(1, D), W2)
    acc2 = _sc_aggregate(src, dst, g2)
    return _t3(acc2, g2, degp)


# speedup vs baseline: 35.3104x; 1.0688x over previous
import functools

import jax
import jax.numpy as jnp
from jax import lax
from jax.experimental import pallas as pl
from jax.experimental.pallas import tpu as pltpu
from jax.experimental.pallas import tpu_sc as plsc

N = 10000
E = 320000
D = 128

NC = 2
NS = 16
CH = 40
E_PER_TILE = E // (NC * NS)
N_CHUNKS = E_PER_TILE // CH
ROWS_PER_TILE = N // NS
ZROWS = 25
OROWS = 40
SD = 25

_mesh = plsc.VectorSubcoreMesh(core_axis_name="c", subcore_axis_name="s")


def _zero_f32_2d(buf, nrows):
    def body(r, carry):
        for j in range(D // 16):
            buf[r, pl.ds(j * 16, 16)] = jnp.zeros((16,), jnp.float32)
        return carry
    lax.fori_loop(0, nrows, body, 0)


R = 8
L = 6
NGRP = (N_CHUNKS + R - 1) // R


@functools.partial(
    pl.kernel,
    mesh=_mesh,
    out_type=jax.ShapeDtypeStruct((NC, N, D), jnp.float32),
    scratch_types=[
        pltpu.VMEM_SHARED((N, D), jnp.float32),
        pltpu.VMEM((3, R, CH), jnp.int32),
        pltpu.VMEM((3, R, CH), jnp.int32),
        pltpu.VMEM((R, CH, D), jnp.float32),
        pltpu.SemaphoreType.DMA,
        *([pltpu.SemaphoreType.DMA] * R),
        *([pltpu.SemaphoreType.DMA] * R),
    ],
)
def _sc_aggregate(src_hbm, dst_hbm, g_hbm, out_hbm, acc, src_blk, dst_blk,
                  rows, isem, *gssems):
    gsems, ssems = gssems[:R], gssems[R:]
    c = lax.axis_index("c")
    s = lax.axis_index("s")

    def zb(r, carry):
        for j in range(D // 16):
            rows[0, r, pl.ds(j * 16, 16)] = jnp.zeros((16,), jnp.float32)
        return carry
    lax.fori_loop(0, ZROWS, zb, 0)
    for t in range(ROWS_PER_TILE // ZROWS):
        pltpu.sync_copy(rows.at[0, pl.ds(0, ZROWS)],
                        acc.at[pl.ds(s * ROWS_PER_TILE + t * ZROWS, ZROWS)])
    plsc.subcore_barrier()

    tile_base = c * (E // NC) + s * E_PER_TILE

    def issue_idx_group(grp, bank):
        for jj in range(R):
            ch = grp * R + jj

            @pl.when(ch < N_CHUNKS)
            def _(jj=jj, ch=ch):
                base = tile_base + ch * CH
                pltpu.async_copy(src_hbm.at[pl.ds(base, CH)],
                                 src_blk.at[bank, jj], isem)
                pltpu.async_copy(dst_hbm.at[pl.ds(base, CH)],
                                 dst_blk.at[bank, jj], isem)

    def drain_idx_group(grp, bank):
        for jj in range(R):
            ch = grp * R + jj

            @pl.when(ch < N_CHUNKS)
            def _(jj=jj, ch=ch):
                base = tile_base + ch * CH
                pltpu.make_async_copy(src_hbm.at[pl.ds(base, CH)],
                                      src_blk.at[bank, jj], isem).wait()
                pltpu.make_async_copy(dst_hbm.at[pl.ds(base, CH)],
                                      dst_blk.at[bank, jj], isem).wait()

    issue_idx_group(0, 0)
    issue_idx_group(1, 1)
    drain_idx_group(0, 0)
    for j in range(L):
        pltpu.async_copy(g_hbm.at[src_blk.at[0, j]], rows.at[j], gsems[j])

    def group_body(G, carry):
        bG = lax.rem(G, 3)
        for j in range(R):
            if j == R - L:
                issue_idx_group(G + 2, lax.rem(G + 2, 3))
                drain_idx_group(G + 1, lax.rem(G + 1, 3))

            a = G * R + j + L
            ra = (j + L) % R
            bank_a = bG if j < R - L else lax.rem(G + 1, 3)

            @pl.when(a < N_CHUNKS)
            def _(j=j, a=a, ra=ra, bank_a=bank_a):
                @pl.when(a >= R)
                def _():
                    pltpu.make_async_copy(rows.at[ra], acc.at[dst_blk.at[0, 0]],
                                          ssems[ra]).wait()
                pltpu.async_copy(g_hbm.at[src_blk.at[bank_a, ra]],
                                 rows.at[ra], gsems[ra])

            c0 = G * R + j

            @pl.when(c0 < N_CHUNKS)
            def _(j=j, c0=c0):
                pltpu.make_async_copy(g_hbm.at[src_blk.at[bG, j]], rows.at[j],
                                      gsems[j]).wait()
                pltpu.async_copy(rows.at[j], acc.at[dst_blk.at[bG, j]],
                                 ssems[j], add=True)
        return carry
    lax.fori_loop(0, NGRP, group_body, 0)

    for r in range(R):
        pltpu.make_async_copy(rows.at[r], acc.at[dst_blk.at[0, 0]],
                              ssems[r]).wait()
    plsc.subcore_barrier()

    @pl.when(s < 10)
    def _():
        for t in range(1000 // OROWS):
            r0 = s * 1000 + t * OROWS
            pltpu.sync_copy(acc.at[pl.ds(r0, OROWS)], rows.at[0])
            pltpu.sync_copy(rows.at[0], out_hbm.at[c, pl.ds(r0, OROWS)])


@functools.partial(
    pl.kernel,
    mesh=_mesh,
    out_type=jax.ShapeDtypeStruct((NC * N,), jnp.float32),
    scratch_types=[
        pltpu.VMEM_SHARED((N,), jnp.float32),
        pltpu.VMEM((SD, CH), jnp.int32),
        pltpu.VMEM((80,), jnp.float32),
        pltpu.VMEM((2000,), jnp.float32),
        pltpu.SemaphoreType.DMA,
        pltpu.SemaphoreType.DMA,
    ],
)
def _sc_degree(dst_hbm, out_hbm, dacc, dst_blk, ones_v, zbuf, isem, ssem):
    c = lax.axis_index("c")
    s = lax.axis_index("s")

    def zbody(i, carry):
        zbuf[pl.ds(i * 16, 16)] = jnp.zeros((16,), jnp.float32)
        return carry
    lax.fori_loop(0, 125, zbody, 0)
    for j in range(5):
        ones_v[pl.ds(j * 16, 16)] = jnp.ones((16,), jnp.float32)

    @pl.when(s == 0)
    def _():
        for t in range(N // 2000):
            pltpu.sync_copy(zbuf, dacc.at[pl.ds(t * 2000, 2000)])
    plsc.subcore_barrier()

    tile_base = c * (E // NC) + s * E_PER_TILE

    def sblock(k, carry):
        base = tile_base + k * (SD * CH)
        hds = [pltpu.async_copy(dst_hbm.at[pl.ds(base + j * CH, CH)],
                                dst_blk.at[j], isem) for j in range(SD)]
        for h in hds:
            h.wait()
        shs = [pltpu.async_copy(ones_v.at[pl.ds(0, CH)], dacc.at[dst_blk.at[j]],
                                ssem, add=True)
               for j in range(SD)]
        for h in shs:
            h.wait()
        return carry
    lax.fori_loop(0, N_CHUNKS // SD, sblock, 0)
    plsc.subcore_barrier()

    @pl.when(s < 10)
    def _():
        pltpu.sync_copy(dacc.at[pl.ds(s * 1000, 1000)], zbuf.at[pl.ds(0, 1000)])
        pltpu.sync_copy(zbuf.at[pl.ds(0, 1000)],
                        out_hbm.at[pl.ds(c * N + s * 1000, 1000)])


BR = 2000


def _dinv_block(degp_ref):
    return lax.rsqrt(degp_ref[0] + degp_ref[1] + 1.0)


def _t0_body(x_ref, w_ref, h_ref):
    h_ref[...] = jnp.dot(x_ref[...], w_ref[...],
                         preferred_element_type=jnp.float32,
                         precision=lax.Precision.DEFAULT)


def _t1_body(h_ref, degp_ref, g_ref):
    g_ref[...] = h_ref[...] * _dinv_block(degp_ref)


def _t2_body(accp_ref, g1_ref, degp_ref, b1_ref, w2_ref, g2_ref):
    dinv = _dinv_block(degp_ref)
    ssum = accp_ref[0] + accp_ref[1] + g1_ref[...]
    z = jnp.maximum(ssum * dinv + b1_ref[...], 0.0)
    h2 = jnp.dot(z, w2_ref[...],
                 preferred_element_type=jnp.float32,
                 precision=lax.Precision.DEFAULT)
    g2_ref[...] = h2 * dinv


def _t3_body(accp_ref, g2_ref, degp_ref, out_ref):
    dinv = _dinv_block(degp_ref)
    y = (accp_ref[0] + accp_ref[1] + g2_ref[...]) * dinv
    m = jnp.max(y, axis=1, keepdims=True)
    lse = jnp.log(jnp.sum(jnp.exp(y - m), axis=1, keepdims=True)) + m
    out_ref[...] = y - lse


_deg_spec = pl.BlockSpec((NC, BR, 1), lambda i: (0, i, 0))
_row_spec = pl.BlockSpec((BR, D), lambda i: (i, 0))
_acc_spec = pl.BlockSpec((NC, BR, D), lambda i: (0, i, 0))
_w_spec = pl.BlockSpec((D, D), lambda i: (0, 0))

_t0 = pl.pallas_call(
    _t0_body,
    grid=(N // BR,),
    in_specs=[_row_spec, _w_spec],
    out_specs=_row_spec,
    out_shape=jax.ShapeDtypeStruct((N, D), jnp.float32),
)

_t1 = pl.pallas_call(
    _t1_body,
    grid=(N // BR,),
    in_specs=[_row_spec, _deg_spec],
    out_specs=_row_spec,
    out_shape=jax.ShapeDtypeStruct((N, D), jnp.float32),
)

_t2 = pl.pallas_call(
    _t2_body,
    grid=(N // BR,),
    in_specs=[_acc_spec, _row_spec, _deg_spec,
              pl.BlockSpec((1, D), lambda i: (0, 0)), _w_spec],
    out_specs=_row_spec,
    out_shape=jax.ShapeDtypeStruct((N, D), jnp.float32),
)

_t3 = pl.pallas_call(
    _t3_body,
    grid=(N // BR,),
    in_specs=[_acc_spec, _row_spec, _deg_spec],
    out_specs=_row_spec,
    out_shape=jax.ShapeDtypeStruct((N, D), jnp.float32),
)


def kernel(x, edge_index, W1, b1, W2):
    src = edge_index[0]
    dst = edge_index[1]
    h1 = _t0(x, W1)
    degp = _sc_degree(dst).reshape(NC, N, 1)
    g1 = _t1(h1, degp)
    acc1 = _sc_aggregate(src, dst, g1)
    g2 = _t2(acc1, g1, degp, b1.reshape(1, D), W2)
    acc2 = _sc_aggregate(src, dst, g2)
    return _t3(acc2, g2, degp)


# speedup vs baseline: 35.3130x; 1.0001x over previous
import functools

import jax
import jax.numpy as jnp
from jax import lax
from jax.experimental import pallas as pl
from jax.experimental.pallas import tpu as pltpu
from jax.experimental.pallas import tpu_sc as plsc

N = 10000
E = 320000
D = 128

NC = 2
NS = 16
CH = 40
E_PER_TILE = E // (NC * NS)
N_CHUNKS = E_PER_TILE // CH
ROWS_PER_TILE = N // NS
ZROWS = 25
OROWS = 40
SD = 25

_mesh = plsc.VectorSubcoreMesh(core_axis_name="c", subcore_axis_name="s")


def _zero_f32_2d(buf, nrows):
    def body(r, carry):
        for j in range(D // 16):
            buf[r, pl.ds(j * 16, 16)] = jnp.zeros((16,), jnp.float32)
        return carry
    lax.fori_loop(0, nrows, body, 0)


R = 8
L = 7
NGRP = (N_CHUNKS + R - 1) // R


@functools.partial(
    pl.kernel,
    mesh=_mesh,
    out_type=jax.ShapeDtypeStruct((NC, N, D), jnp.float32),
    scratch_types=[
        pltpu.VMEM_SHARED((N, D), jnp.float32),
        pltpu.VMEM((3, R, CH), jnp.int32),
        pltpu.VMEM((3, R, CH), jnp.int32),
        pltpu.VMEM((R, CH, D), jnp.float32),
        pltpu.SemaphoreType.DMA,
        *([pltpu.SemaphoreType.DMA] * R),
        *([pltpu.SemaphoreType.DMA] * R),
    ],
)
def _sc_aggregate(src_hbm, dst_hbm, g_hbm, out_hbm, acc, src_blk, dst_blk,
                  rows, isem, *gssems):
    gsems, ssems = gssems[:R], gssems[R:]
    c = lax.axis_index("c")
    s = lax.axis_index("s")

    def zb(r, carry):
        for j in range(D // 16):
            rows[0, r, pl.ds(j * 16, 16)] = jnp.zeros((16,), jnp.float32)
        return carry
    lax.fori_loop(0, ZROWS, zb, 0)
    for t in range(ROWS_PER_TILE // ZROWS):
        pltpu.sync_copy(rows.at[0, pl.ds(0, ZROWS)],
                        acc.at[pl.ds(s * ROWS_PER_TILE + t * ZROWS, ZROWS)])
    plsc.subcore_barrier()

    tile_base = c * (E // NC) + s * E_PER_TILE

    def issue_idx_group(grp, bank):
        for jj in range(R):
            ch = grp * R + jj

            @pl.when(ch < N_CHUNKS)
            def _(jj=jj, ch=ch):
                base = tile_base + ch * CH
                pltpu.async_copy(src_hbm.at[pl.ds(base, CH)],
                                 src_blk.at[bank, jj], isem)
                pltpu.async_copy(dst_hbm.at[pl.ds(base, CH)],
                                 dst_blk.at[bank, jj], isem)

    def drain_idx_group(grp, bank):
        for jj in range(R):
            ch = grp * R + jj

            @pl.when(ch < N_CHUNKS)
            def _(jj=jj, ch=ch):
                base = tile_base + ch * CH
                pltpu.make_async_copy(src_hbm.at[pl.ds(base, CH)],
                                      src_blk.at[bank, jj], isem).wait()
                pltpu.make_async_copy(dst_hbm.at[pl.ds(base, CH)],
                                      dst_blk.at[bank, jj], isem).wait()

    issue_idx_group(0, 0)
    issue_idx_group(1, 1)
    drain_idx_group(0, 0)
    for j in range(L):
        pltpu.async_copy(g_hbm.at[src_blk.at[0, j]], rows.at[j], gsems[j])

    def group_body(G, carry):
        bG = lax.rem(G, 3)
        for j in range(R):
            if j == R - L:
                issue_idx_group(G + 2, lax.rem(G + 2, 3))
                drain_idx_group(G + 1, lax.rem(G + 1, 3))

            a = G * R + j + L
            ra = (j + L) % R
            bank_a = bG if j < R - L else lax.rem(G + 1, 3)

            @pl.when(a < N_CHUNKS)
            def _(j=j, a=a, ra=ra, bank_a=bank_a):
                @pl.when(a >= R)
                def _():
                    pltpu.make_async_copy(rows.at[ra], acc.at[dst_blk.at[0, 0]],
                                          ssems[ra]).wait()
                pltpu.async_copy(g_hbm.at[src_blk.at[bank_a, ra]],
                                 rows.at[ra], gsems[ra])

            c0 = G * R + j

            @pl.when(c0 < N_CHUNKS)
            def _(j=j, c0=c0):
                pltpu.make_async_copy(g_hbm.at[src_blk.at[bG, j]], rows.at[j],
                                      gsems[j]).wait()
                pltpu.async_copy(rows.at[j], acc.at[dst_blk.at[bG, j]],
                                 ssems[j], add=True)
        return carry
    lax.fori_loop(0, NGRP, group_body, 0)

    for r in range(R):
        pltpu.make_async_copy(rows.at[r], acc.at[dst_blk.at[0, 0]],
                              ssems[r]).wait()
    plsc.subcore_barrier()

    @pl.when(s < 10)
    def _():
        for t in range(1000 // OROWS):
            r0 = s * 1000 + t * OROWS
            pltpu.sync_copy(acc.at[pl.ds(r0, OROWS)], rows.at[0])
            pltpu.sync_copy(rows.at[0], out_hbm.at[c, pl.ds(r0, OROWS)])


@functools.partial(
    pl.kernel,
    mesh=_mesh,
    out_type=jax.ShapeDtypeStruct((NC * N,), jnp.float32),
    scratch_types=[
        pltpu.VMEM_SHARED((N,), jnp.float32),
        pltpu.VMEM((SD, CH), jnp.int32),
        pltpu.VMEM((80,), jnp.float32),
        pltpu.VMEM((2000,), jnp.float32),
        pltpu.SemaphoreType.DMA,
        pltpu.SemaphoreType.DMA,
    ],
)
def _sc_degree(dst_hbm, out_hbm, dacc, dst_blk, ones_v, zbuf, isem, ssem):
    c = lax.axis_index("c")
    s = lax.axis_index("s")

    def zbody(i, carry):
        zbuf[pl.ds(i * 16, 16)] = jnp.zeros((16,), jnp.float32)
        return carry
    lax.fori_loop(0, 125, zbody, 0)
    for j in range(5):
        ones_v[pl.ds(j * 16, 16)] = jnp.ones((16,), jnp.float32)

    @pl.when(s == 0)
    def _():
        for t in range(N // 2000):
            pltpu.sync_copy(zbuf, dacc.at[pl.ds(t * 2000, 2000)])
    plsc.subcore_barrier()

    tile_base = c * (E // NC) + s * E_PER_TILE

    def sblock(k, carry):
        base = tile_base + k * (SD * CH)
        hds = [pltpu.async_copy(dst_hbm.at[pl.ds(base + j * CH, CH)],
                                dst_blk.at[j], isem) for j in range(SD)]
        for h in hds:
            h.wait()
        shs = [pltpu.async_copy(ones_v.at[pl.ds(0, CH)], dacc.at[dst_blk.at[j]],
                                ssem, add=True)
               for j in range(SD)]
        for h in shs:
            h.wait()
        return carry
    lax.fori_loop(0, N_CHUNKS // SD, sblock, 0)
    plsc.subcore_barrier()

    @pl.when(s < 10)
    def _():
        pltpu.sync_copy(dacc.at[pl.ds(s * 1000, 1000)], zbuf.at[pl.ds(0, 1000)])
        pltpu.sync_copy(zbuf.at[pl.ds(0, 1000)],
                        out_hbm.at[pl.ds(c * N + s * 1000, 1000)])


BR = 2000


def _dinv_block(degp_ref):
    return lax.rsqrt(degp_ref[0] + degp_ref[1] + 1.0)


def _t0_body(x_ref, w_ref, h_ref):
    h_ref[...] = jnp.dot(x_ref[...], w_ref[...],
                         preferred_element_type=jnp.float32,
                         precision=lax.Precision.DEFAULT)


def _t1_body(h_ref, degp_ref, g_ref):
    g_ref[...] = h_ref[...] * _dinv_block(degp_ref)


def _t2_body(accp_ref, g1_ref, degp_ref, b1_ref, w2_ref, g2_ref):
    dinv = _dinv_block(degp_ref)
    ssum = accp_ref[0] + accp_ref[1] + g1_ref[...]
    z = jnp.maximum(ssum * dinv + b1_ref[...], 0.0)
    h2 = jnp.dot(z, w2_ref[...],
                 preferred_element_type=jnp.float32,
                 precision=lax.Precision.DEFAULT)
    g2_ref[...] = h2 * dinv


def _t3_body(accp_ref, g2_ref, degp_ref, out_ref):
    dinv = _dinv_block(degp_ref)
    y = (accp_ref[0] + accp_ref[1] + g2_ref[...]) * dinv
    m = jnp.max(y, axis=1, keepdims=True)
    lse = jnp.log(jnp.sum(jnp.exp(y - m), axis=1, keepdims=True)) + m
    out_ref[...] = y - lse


_deg_spec = pl.BlockSpec((NC, BR, 1), lambda i: (0, i, 0))
_row_spec = pl.BlockSpec((BR, D), lambda i: (i, 0))
_acc_spec = pl.BlockSpec((NC, BR, D), lambda i: (0, i, 0))
_w_spec = pl.BlockSpec((D, D), lambda i: (0, 0))

_t0 = pl.pallas_call(
    _t0_body,
    grid=(N // BR,),
    in_specs=[_row_spec, _w_spec],
    out_specs=_row_spec,
    out_shape=jax.ShapeDtypeStruct((N, D), jnp.float32),
)

_t1 = pl.pallas_call(
    _t1_body,
    grid=(N // BR,),
    in_specs=[_row_spec, _deg_spec],
    out_specs=_row_spec,
    out_shape=jax.ShapeDtypeStruct((N, D), jnp.float32),
)

_t2 = pl.pallas_call(
    _t2_body,
    grid=(N // BR,),
    in_specs=[_acc_spec, _row_spec, _deg_spec,
              pl.BlockSpec((1, D), lambda i: (0, 0)), _w_spec],
    out_specs=_row_spec,
    out_shape=jax.ShapeDtypeStruct((N, D), jnp.float32),
)

_t3 = pl.pallas_call(
    _t3_body,
    grid=(N // BR,),
    in_specs=[_acc_spec, _row_spec, _deg_spec],
    out_specs=_row_spec,
    out_shape=jax.ShapeDtypeStruct((N, D), jnp.float32),
)


def kernel(x, edge_index, W1, b1, W2):
    src = edge_index[0]
    dst = edge_index[1]
    h1 = _t0(x, W1)
    degp = _sc_degree(dst).reshape(NC, N, 1)
    g1 = _t1(h1, degp)
    acc1 = _sc_aggregate(src, dst, g1)
    g2 = _t2(acc1, g1, degp, b1.reshape(1, D), W2)
    acc2 = _sc_aggregate(src, dst, g2)
    return _t3(acc2, g2, degp)
